# Initial kernel scaffold; baseline (speedup 1.0000x reference)
import jax
import jax.numpy as jnp
from jax.experimental import pallas as pl


def kernel(hidden_states, gate_w, w1, w2, w3):
    raise NotImplementedError("write your pallas kernel here")


# speedup vs baseline: 1.3339x; 1.3339x over previous
import functools

import jax
import jax.numpy as jnp
from jax.experimental import pallas as pl
from jax.experimental.pallas import tpu as pltpu

_JITTER = 0.01
_EPAD = 128


def _routing_body(x_ref, gw_ref, logits_ref, wfull_ref, *, n_experts):
    x = x_ref[...]
    logits = jax.lax.dot_general(
        x, gw_ref[...], (((1,), (1,)), ((), ())),
        preferred_element_type=jnp.float32)
    logits_ref[...] = logits
    col = jax.lax.broadcasted_iota(jnp.int32, logits.shape, 1)
    valid = col < n_experts
    neg_inf = jnp.float32(-jnp.inf)
    scores = jnp.where(valid, logits, neg_inf)

    m1 = jnp.max(scores, axis=1, keepdims=True)
    i1 = jnp.argmax(scores, axis=1)[:, None]
    factor1 = jnp.maximum(jnp.abs(scores), m1)
    mask1 = (m1 - scores) / factor1 > 2.0 * _JITTER
    masked1 = jnp.where(mask1, neg_inf, scores)
    e1 = jnp.exp(masked1 - m1)
    p1 = e1 / jnp.sum(e1, axis=1, keepdims=True)
    sel1 = col == i1
    mult1 = jnp.sum(jnp.where(sel1, p1, 0.0), axis=1, keepdims=True)

    scores2 = jnp.where(sel1, neg_inf, scores)
    m2 = jnp.max(scores2, axis=1, keepdims=True)
    i2 = jnp.argmax(scores2, axis=1)[:, None]
    factor2 = jnp.maximum(jnp.abs(scores), m2)
    mask2 = (m2 - scores) / factor2 > 2.0 * _JITTER
    masked2 = jnp.where(mask2, neg_inf, scores2)
    e2 = jnp.exp(masked2 - m2)
    p2 = e2 / jnp.sum(e2, axis=1, keepdims=True)
    sel2 = col == i2
    mult2 = jnp.sum(jnp.where(sel2, p2, 0.0), axis=1, keepdims=True)

    wfull_ref[...] = jnp.where(sel1, mult1, 0.0) + jnp.where(sel2, mult2, 0.0)


def _moe_body(x_ref, w1_ref, w3_ref, w2_ref, wf_ref, out_ref, *, nsteps_e,
              nsteps_f):
    e = pl.program_id(0)
    nf = pl.program_id(1)
    x = x_ref[...]
    g = jax.lax.dot_general(x, w1_ref[0], (((1,), (1,)), ((), ())),
                            preferred_element_type=jnp.float32)
    u = jax.lax.dot_general(x, w3_ref[0], (((1,), (1,)), ((), ())),
                            preferred_element_type=jnp.float32)
    a = (g * jax.lax.logistic(g)) * u
    partial = jax.lax.dot_general(a, w2_ref[0], (((1,), (1,)), ((), ())),
                                  preferred_element_type=jnp.float32)
    wf = wf_ref[...]
    col = jax.lax.broadcasted_iota(jnp.int32, wf.shape, 1)
    wcol = jnp.sum(jnp.where(col == e, wf, 0.0), axis=1, keepdims=True)
    contrib = wcol * partial

    @pl.when(jnp.logical_and(e == 0, nf == 0))
    def _():
        out_ref[...] = contrib

    @pl.when(jnp.logical_or(e != 0, nf != 0))
    def _():
        out_ref[...] = out_ref[...] + contrib


def kernel(hidden_states, gate_w, w1, w2, w3):
    b, s, h = hidden_states.shape
    t = b * s
    n_experts, ffn, _ = w1.shape
    x = hidden_states.reshape(t, h)

    gw_pad = jnp.zeros((_EPAD, h), jnp.float32).at[:n_experts].set(gate_w)

    bm_r = min(256, t)
    logits_pad, wfull = pl.pallas_call(
        functools.partial(_routing_body, n_experts=n_experts),
        grid=(t // bm_r,),
        in_specs=[
            pl.BlockSpec((bm_r, h), lambda i: (i, 0)),
            pl.BlockSpec((_EPAD, h), lambda i: (0, 0)),
        ],
        out_specs=[
            pl.BlockSpec((bm_r, _EPAD), lambda i: (i, 0)),
            pl.BlockSpec((bm_r, _EPAD), lambda i: (i, 0)),
        ],
        out_shape=[
            jax.ShapeDtypeStruct((t, _EPAD), jnp.float32),
            jax.ShapeDtypeStruct((t, _EPAD), jnp.float32),
        ],
    )(x, gw_pad)
    router_logits = logits_pad[:, :n_experts]

    bf = 512 if ffn % 512 == 0 else ffn
    nsteps_f = ffn // bf
    out = pl.pallas_call(
        functools.partial(_moe_body, nsteps_e=n_experts, nsteps_f=nsteps_f),
        grid=(n_experts, nsteps_f),
        in_specs=[
            pl.BlockSpec((t, h), lambda e, f: (0, 0)),
            pl.BlockSpec((1, bf, h), lambda e, f: (e, f, 0)),
            pl.BlockSpec((1, bf, h), lambda e, f: (e, f, 0)),
            pl.BlockSpec((1, h, bf), lambda e, f: (e, 0, f)),
            pl.BlockSpec((t, _EPAD), lambda e, f: (0, 0)),
        ],
        out_specs=pl.BlockSpec((t, h), lambda e, f: (0, 0)),
        out_shape=jax.ShapeDtypeStruct((t, h), jnp.float32),
        compiler_params=pltpu.CompilerParams(
            dimension_semantics=("arbitrary", "arbitrary")),
    )(x, w1, w3, w2, wfull)

    return out.reshape(b, s, h), router_logits
